# baseline (speedup 1.0000x reference)
import functools

import jax
import jax.numpy as jnp
from jax import lax
from jax.experimental import pallas as pl
from jax.experimental.pallas import tpu as pltpu
from jax.experimental.pallas import tpu_sc as plsc

NUM_FIELDS = 26
VOCAB = 100000
EMB_DIM = 32
BATCH = 16384

NW = 32
LANES = 16
B_PER_W = BATCH // NW
B_CHUNK = 64
N_CHUNKS = B_PER_W // B_CHUNK
CHUNK_IDX = B_CHUNK * NUM_FIELDS
PAD = 32

_mesh = plsc.VectorSubcoreMesh(core_axis_name="c", subcore_axis_name="s")


@functools.partial(
    pl.kernel,
    out_type=jax.ShapeDtypeStruct((BATCH, NUM_FIELDS, EMB_DIM), jnp.float32),
    mesh=_mesh,
    scratch_types=[
        pltpu.VMEM((CHUNK_IDX,), jnp.int32),
        pltpu.VMEM((B_CHUNK * PAD,), jnp.int32),
        pltpu.VMEM((B_CHUNK, NUM_FIELDS, EMB_DIM), jnp.float32),
        pltpu.SemaphoreType.DMA,
    ],
    compiler_params=pltpu.CompilerParams(
        use_tc_tiling_on_sc=False, needs_layout_passes=False
    ),
)
def _gather_kernel(tables_hbm, x_hbm, out_hbm, xraw_v, idxp_v, rows_v, sem):
    wid = lax.axis_index("s") * 2 + lax.axis_index("c")
    b_base = wid * B_PER_W

    fld0 = jnp.minimum(lax.iota(jnp.int32, LANES), NUM_FIELDS - 1)
    fld1 = jnp.minimum(lax.iota(jnp.int32, LANES) + LANES, NUM_FIELDS - 1)

    def _chunk_body(c, carry):
        b0 = b_base + c * B_CHUNK
        pltpu.sync_copy(x_hbm.at[pl.ds(b0 * NUM_FIELDS, CHUNK_IDX)], xraw_v)

        def _prep_body(t, carry2):
            i = t // 2
            k = t % 2
            fld = jnp.where(k == 0, fld0, fld1)
            src = plsc.load_gather(xraw_v, [i * NUM_FIELDS + fld])
            idxp_v[pl.ds(i * PAD + k * LANES, LANES)] = src + fld * VOCAB
            return carry2

        lax.fori_loop(0, B_CHUNK * 2, _prep_body, 0)

        def _fire_body(i, carry2):
            pltpu.async_copy(
                tables_hbm.at[idxp_v.at[pl.ds(i * PAD, NUM_FIELDS)]],
                rows_v.at[i],
                sem,
            )
            return carry2

        lax.fori_loop(0, B_CHUNK, _fire_body, 0)
        pltpu.make_async_copy(out_hbm.at[pl.ds(0, B_CHUNK)], rows_v, sem).wait()

        pltpu.sync_copy(rows_v, out_hbm.at[pl.ds(b0, B_CHUNK)])
        return carry

    lax.fori_loop(0, N_CHUNKS, _chunk_body, 0)


def kernel(x, tables):
    x_flat = x.astype(jnp.int32).reshape(BATCH * NUM_FIELDS)
    tables_flat = tables.reshape(NUM_FIELDS * VOCAB, EMB_DIM)
    out = _gather_kernel(tables_flat, x_flat)
    return out.reshape(BATCH, NUM_FIELDS * EMB_DIM)


# speedup vs baseline: 1.7233x; 1.7233x over previous
import functools

import jax
import jax.numpy as jnp
from jax import lax
from jax.experimental import pallas as pl
from jax.experimental.pallas import tpu as pltpu
from jax.experimental.pallas import tpu_sc as plsc

NUM_FIELDS = 26
VOCAB = 100000
EMB_DIM = 32
BATCH = 16384

N_TROWS = NUM_FIELDS * EMB_DIM
NW = 32
LANES = 16
R_PER_W = N_TROWS // NW
OUT_CHUNK = 8192
N_OCHUNK = BATCH // OUT_CHUNK

_mesh = plsc.VectorSubcoreMesh(core_axis_name="c", subcore_axis_name="s")


@functools.partial(
    pl.kernel,
    out_type=jax.ShapeDtypeStruct((N_TROWS, BATCH), jnp.float32),
    mesh=_mesh,
    scratch_types=[
        pltpu.VMEM((VOCAB,), jnp.float32),
        pltpu.VMEM((BATCH,), jnp.int32),
        pltpu.VMEM((OUT_CHUNK,), jnp.float32),
        pltpu.SemaphoreType.DMA,
    ],
    compiler_params=pltpu.CompilerParams(
        use_tc_tiling_on_sc=False, needs_layout_passes=False
    ),
)
def _gather_kernel(tt_hbm, xt_hbm, out_hbm, row_v, xb_v, ob_v, sem):
    wid = lax.axis_index("s") * 2 + lax.axis_index("c")
    r_base = wid * R_PER_W

    def _row_body(i_r, f_prev):
        r = r_base + i_r
        f = r // EMB_DIM

        @pl.when(f != f_prev)
        def _load_x():
            pltpu.sync_copy(xt_hbm.at[f], xb_v)

        pltpu.sync_copy(tt_hbm.at[r], row_v)

        def _ochunk_body(oc, carry2):
            def _vec_body(i, carry3):
                xv = xb_v[pl.ds(oc * OUT_CHUNK + i * LANES, LANES)]
                ob_v[pl.ds(i * LANES, LANES)] = plsc.load_gather(row_v, [xv])
                return carry3

            lax.fori_loop(0, OUT_CHUNK // LANES, _vec_body, 0)
            pltpu.sync_copy(ob_v, out_hbm.at[r, pl.ds(oc * OUT_CHUNK, OUT_CHUNK)])
            return carry2

        lax.fori_loop(0, N_OCHUNK, _ochunk_body, 0)
        return f

    lax.fori_loop(0, R_PER_W, _row_body, jnp.int32(-1))


def kernel(x, tables):
    tt = tables.transpose(0, 2, 1).reshape(N_TROWS, VOCAB)
    xt = x.astype(jnp.int32).T
    out_t = _gather_kernel(tt, xt)
    return out_t.T.reshape(BATCH, NUM_FIELDS * EMB_DIM)


# speedup vs baseline: 2.0339x; 1.1802x over previous
import functools

import jax
import jax.numpy as jnp
from jax import lax
from jax.experimental import pallas as pl
from jax.experimental.pallas import tpu as pltpu
from jax.experimental.pallas import tpu_sc as plsc

NUM_FIELDS = 26
VOCAB = 100000
EMB_DIM = 32
BATCH = 16384

N_TROWS = NUM_FIELDS * EMB_DIM
NW = 32
LANES = 16
R_PER_W = N_TROWS // NW
OUT_CHUNK = 8192
N_OCHUNK = BATCH // OUT_CHUNK

_mesh = plsc.VectorSubcoreMesh(core_axis_name="c", subcore_axis_name="s")


@functools.partial(
    pl.kernel,
    out_type=jax.ShapeDtypeStruct((N_TROWS, BATCH), jnp.float32),
    mesh=_mesh,
    scratch_types=[
        pltpu.VMEM((VOCAB,), jnp.float32),
        pltpu.VMEM((BATCH,), jnp.int32),
        pltpu.VMEM((OUT_CHUNK,), jnp.float32),
        pltpu.SemaphoreType.DMA,
    ],
    compiler_params=pltpu.CompilerParams(
        use_tc_tiling_on_sc=False, needs_layout_passes=False
    ),
)
def _gather_kernel(tt_hbm, xt_hbm, out_hbm, row_v, xb_v, ob_v, sem):
    wid = lax.axis_index("s") * 2 + lax.axis_index("c")
    r_base = wid * R_PER_W

    def _row_body(i_r, f_prev):
        r = r_base + i_r
        f = r // EMB_DIM

        @pl.when(f != f_prev)
        def _load_x():
            pltpu.sync_copy(xt_hbm.at[f], xb_v)

        pltpu.sync_copy(tt_hbm.at[r], row_v)

        def _ochunk_body(oc, carry2):
            @plsc.parallel_loop(0, OUT_CHUNK // LANES, unroll=8)
            def _vec_body(i):
                xv = xb_v[pl.ds(oc * OUT_CHUNK + i * LANES, LANES)]
                ob_v[pl.ds(i * LANES, LANES)] = plsc.load_gather(row_v, [xv])

            pltpu.sync_copy(ob_v, out_hbm.at[r, pl.ds(oc * OUT_CHUNK, OUT_CHUNK)])
            return carry2

        lax.fori_loop(0, N_OCHUNK, _ochunk_body, 0)
        return f

    lax.fori_loop(0, R_PER_W, _row_body, jnp.int32(-1))


def kernel(x, tables):
    tt = tables.transpose(0, 2, 1).reshape(N_TROWS, VOCAB)
    xt = x.astype(jnp.int32).T
    out_t = _gather_kernel(tt, xt)
    return out_t.T.reshape(BATCH, NUM_FIELDS * EMB_DIM)
